# baseline (speedup 1.0000x reference)
import functools

import jax
import jax.numpy as jnp
from jax import lax
from jax.experimental import pallas as pl
from jax.experimental.pallas import tpu as pltpu
from jax.experimental.pallas import tpu_sc as plsc

L = 1024
E = 8192
NC = 2
NS = 16
EPW = E // (NC * NS)
WPS = (L * L) // NS
ZCH = 8192


def _sc_counts(src, dst):
    mesh = plsc.VectorSubcoreMesh(core_axis_name="c", subcore_axis_name="s")

    @functools.partial(
        pl.kernel,
        mesh=mesh,
        out_type=jax.ShapeDtypeStruct((NC, L * L), jnp.float32),
        scratch_types=[
            pltpu.VMEM((EPW,), jnp.int32),
            pltpu.VMEM((EPW,), jnp.int32),
            pltpu.VMEM((EPW // 128, 128), jnp.int32),
            pltpu.VMEM((128,), jnp.float32),
            pltpu.VMEM((ZCH,), jnp.float32),
            pltpu.VMEM_SHARED((L * L,), jnp.float32),
        ],
    )
    def k(src_hbm, dst_hbm, out_hbm, sv, dv, iv, ones_v, zv, csh):
        cid = lax.axis_index("c")
        sid = lax.axis_index("s")

        def fill16(i, ref, val):
            ref[pl.ds(i * 16, 16)] = jnp.full((16,), val, ref.dtype)

        lax.fori_loop(0, ZCH // 16, lambda i, c: (fill16(i, zv, 0.0), c)[1], 0)
        lax.fori_loop(0, 128 // 16, lambda i, c: (fill16(i, ones_v, 1.0), c)[1], 0)

        base = sid * WPS

        def zc(j, c):
            pltpu.sync_copy(zv, csh.at[pl.ds(base + j * ZCH, ZCH)])
            return c

        lax.fori_loop(0, WPS // ZCH, zc, 0)
        plsc.subcore_barrier()

        ebase = (cid * NS + sid) * EPW
        pltpu.sync_copy(src_hbm.at[pl.ds(ebase, EPW)], sv)
        pltpu.sync_copy(dst_hbm.at[pl.ds(ebase, EPW)], dv)
        for g in range(EPW // 128):
            for j in range(128 // 16):
                s16 = sv[pl.ds(g * 128 + j * 16, 16)]
                d16 = dv[pl.ds(g * 128 + j * 16, 16)]
                iv[g, pl.ds(j * 16, 16)] = s16 * L + d16

        for g in range(EPW // 128):
            pltpu.sync_copy(ones_v, csh.at[iv.at[g]], add=True)
        plsc.subcore_barrier()

        def co(j, c):
            pltpu.sync_copy(csh.at[pl.ds(base + j * ZCH, ZCH)],
                            out_hbm.at[cid, pl.ds(base + j * ZCH, ZCH)])
            return c

        lax.fori_loop(0, WPS // ZCH, co, 0)

    return k(src, dst)


def _prep_body(ct2_ref, a_ref):
    ct = ct2_ref[0] + ct2_ref[1]
    deg = jnp.sum(ct, axis=0, keepdims=True) + 2.0
    di = lax.rsqrt(deg)
    eyem = jnp.where(
        lax.broadcasted_iota(jnp.int32, (L, L), 0)
        == lax.broadcasted_iota(jnp.int32, (L, L), 1),
        1.0, 0.0)
    a = jnp.dot(eyem * di, ct * di, preferred_element_type=jnp.float32,
                precision=lax.Precision.HIGHEST)
    a_ref[...] = (a + eyem * (2.0 * di * di)).astype(jnp.bfloat16)


def _prep(counts2):
    return pl.pallas_call(
        _prep_body,
        out_shape=jax.ShapeDtypeStruct((L, L), jnp.bfloat16),
    )(counts2)


def _stage_body(G, B, out_dtype, x_ref, a_ref,
                wt_ref, b_ref, g_ref, be_ref,
                wst_ref, bs_ref, gs_ref, bes_ref,
                out_ref, z_ref, zs_ref, ts_ref):
    cout = out_ref.shape[1]
    n = G // B
    a = a_ref[...]
    wt = wt_ref[...].astype(jnp.bfloat16)
    wst = wst_ref[...].astype(jnp.bfloat16)
    b = b_ref[...]
    bs = bs_ref[...]

    def body1(g, carry):
        ssum, ssq = carry
        t = jnp.dot(x_ref[g], a, preferred_element_type=jnp.float32)
        bb = g // n

        @pl.when(g % n == 0)
        def _():
            ts_ref[bb] = t

        @pl.when(g % n != 0)
        def _():
            ts_ref[bb] = ts_ref[bb] + t

        z = jnp.dot(wt, t.astype(jnp.bfloat16),
                    preferred_element_type=jnp.float32) + b
        z_ref[g] = z
        return (ssum + jnp.sum(z, axis=1, keepdims=True),
                ssq + jnp.sum(z * z, axis=1, keepdims=True))

    zc = jnp.zeros((cout, 1), jnp.float32)
    ssum, ssq = lax.fori_loop(0, G, body1, (zc, zc))
    mean = ssum / (G * L)
    var = ssq / (G * L) - mean * mean
    rstd = lax.rsqrt(var + 1e-5)

    def body2(bb, carry):
        ssum, ssq = carry
        z2 = jnp.dot(wst, ts_ref[bb].astype(jnp.bfloat16),
                     preferred_element_type=jnp.float32) + bs
        zs_ref[bb] = z2
        return (ssum + jnp.sum(z2, axis=1, keepdims=True),
                ssq + jnp.sum(z2 * z2, axis=1, keepdims=True))

    s2sum, s2sq = lax.fori_loop(0, B, body2, (zc, zc))
    mean2 = s2sum / (B * L)
    var2 = s2sq / (B * L) - mean2 * mean2
    rstd2 = lax.rsqrt(var2 + 1e-5)

    sc1 = rstd * g_ref[...]
    of1 = be_ref[...] - mean * sc1
    sc2 = rstd2 * gs_ref[...]
    of2 = bes_ref[...] - mean2 * sc2

    def body3(g, c):
        z1 = z_ref[g] * sc1 + of1
        z2 = zs_ref[g // n] * sc2 + of2
        out_ref[g] = jnp.maximum(z1 + z2, 0.0).astype(out_dtype)
        return c

    lax.fori_loop(0, G, body3, 0)


def _stage(x, a, wt, b, gam, bet, wst, bs, gs, bes, out_dtype):
    cout = wt.shape[0]
    cin = wt.shape[1]
    G = x.shape[0]
    B = 4
    return pl.pallas_call(
        functools.partial(_stage_body, G, B, out_dtype),
        out_shape=jax.ShapeDtypeStruct((G, cout, L), out_dtype),
        scratch_shapes=[pltpu.VMEM((G, cout, L), jnp.float32),
                        pltpu.VMEM((B, cout, L), jnp.float32),
                        pltpu.VMEM((B, cin, L), jnp.float32)],
    )(x, a, wt, b, gam, bet, wst, bs, gs, bes)


def _col(v):
    return v.reshape(-1, 1)


def kernel(x, edge_index, W1, b1, g1, be1, W1s, b1s, g1s, be1s,
           W2, b2, g2, be2, W2s, b2s, g2s, be2s,
           W3, b3, g3, be3, W3s, b3s, g3s, be3s):
    ei = edge_index.astype(jnp.int32)
    counts2 = _sc_counts(ei[0], ei[1])
    a = _prep(counts2.reshape(NC, L, L))

    h = x.reshape(16, x.shape[2], L).astype(jnp.bfloat16)
    h = _stage(h, a, W1.T, _col(b1), _col(g1), _col(be1),
               W1s.T, _col(b1s), _col(g1s), _col(be1s), jnp.bfloat16)
    h = _stage(h, a, W2.T, _col(b2), _col(g2), _col(be2),
               W2s.T, _col(b2s), _col(g2s), _col(be2s), jnp.bfloat16)
    h = _stage(h, a, W3.T, _col(b3), _col(g3), _col(be3),
               W3s.T, _col(b3s), _col(g3s), _col(be3s), jnp.float32)
    return h


# speedup vs baseline: 1.0838x; 1.0838x over previous
import functools

import jax
import jax.numpy as jnp
from jax import lax
from jax.experimental import pallas as pl
from jax.experimental.pallas import tpu as pltpu
from jax.experimental.pallas import tpu_sc as plsc

L = 1024
E = 8192
NC = 2
NS = 16
EPW = E // (NC * NS)
WPS = (L * L) // NS
ZCH = 8192


def _sc_counts(src, dst):
    mesh = plsc.VectorSubcoreMesh(core_axis_name="c", subcore_axis_name="s")

    @functools.partial(
        pl.kernel,
        mesh=mesh,
        out_type=(jax.ShapeDtypeStruct((NC, L * L), jnp.float32),
                  jax.ShapeDtypeStruct((NC, L), jnp.float32)),
        scratch_types=[
            pltpu.VMEM((EPW,), jnp.int32),
            pltpu.VMEM((EPW // 128, 128), jnp.int32),
            pltpu.VMEM((EPW // 128, 128), jnp.int32),
            pltpu.VMEM((128,), jnp.float32),
            pltpu.VMEM((ZCH,), jnp.float32),
            pltpu.VMEM_SHARED((L * L,), jnp.float32),
            pltpu.VMEM_SHARED((L,), jnp.float32),
        ],
    )
    def k(src_hbm, dst_hbm, out_hbm, deg_hbm, sv, dv, iv, ones_v, zv, csh,
          dsh):
        cid = lax.axis_index("c")
        sid = lax.axis_index("s")

        def fill16(i, ref, val):
            ref[pl.ds(i * 16, 16)] = jnp.full((16,), val, ref.dtype)

        lax.fori_loop(0, ZCH // 16, lambda i, c: (fill16(i, zv, 0.0), c)[1], 0)
        lax.fori_loop(0, 128 // 16, lambda i, c: (fill16(i, ones_v, 1.0), c)[1], 0)

        base = sid * WPS

        def zc(j, c):
            pltpu.sync_copy(zv, csh.at[pl.ds(base + j * ZCH, ZCH)])
            return c

        lax.fori_loop(0, WPS // ZCH, zc, 0)

        @pl.when(sid == 0)
        def _():
            pltpu.sync_copy(zv.at[pl.ds(0, L)], dsh)

        plsc.subcore_barrier()

        ebase = (cid * NS + sid) * EPW
        pltpu.sync_copy(src_hbm.at[pl.ds(ebase, EPW)], sv)
        for g in range(EPW // 128):
            pltpu.sync_copy(dst_hbm.at[pl.ds(ebase + g * 128, 128)],
                            dv.at[g])
        for g in range(EPW // 128):
            for j in range(128 // 16):
                s16 = sv[pl.ds(g * 128 + j * 16, 16)]
                d16 = dv[g, pl.ds(j * 16, 16)]
                iv[g, pl.ds(j * 16, 16)] = s16 * L + d16

        for g in range(EPW // 128):
            pltpu.sync_copy(ones_v, csh.at[iv.at[g]], add=True)
            pltpu.sync_copy(ones_v, dsh.at[dv.at[g]], add=True)
        plsc.subcore_barrier()

        def co(j, c):
            pltpu.sync_copy(csh.at[pl.ds(base + j * ZCH, ZCH)],
                            out_hbm.at[cid, pl.ds(base + j * ZCH, ZCH)])
            return c

        lax.fori_loop(0, WPS // ZCH, co, 0)

        @pl.when(sid == 0)
        def _():
            pltpu.sync_copy(dsh, deg_hbm.at[cid])

    return k(src, dst)


def _prep_body(ct2_ref, degp_ref, a_ref):
    ct = ct2_ref[0] + ct2_ref[1]
    degc = degp_ref[0] + degp_ref[1] + 2.0
    dic = lax.rsqrt(degc)
    degr = jnp.sum(ct, axis=0, keepdims=True) + 2.0
    dir_ = lax.rsqrt(degr)
    eyem = jnp.where(
        lax.broadcasted_iota(jnp.int32, (L, L), 0)
        == lax.broadcasted_iota(jnp.int32, (L, L), 1),
        1.0, 0.0)
    a = dic * ct * dir_ + eyem * (2.0 * dir_ * dir_)
    a_ref[...] = a.astype(jnp.bfloat16)


def _prep(counts2, degp):
    return pl.pallas_call(
        _prep_body,
        out_shape=jax.ShapeDtypeStruct((L, L), jnp.bfloat16),
    )(counts2, degp)


def _stage_body(G, B, out_dtype, x_ref, a_ref,
                wt_ref, b_ref, g_ref, be_ref,
                wst_ref, bs_ref, gs_ref, bes_ref,
                out_ref, z_ref, zs_ref, ts_ref):
    cout = out_ref.shape[1]
    n = G // B
    a = a_ref[...]
    wt = wt_ref[...].astype(jnp.bfloat16)
    wst = wst_ref[...].astype(jnp.bfloat16)
    b = b_ref[...]
    bs = bs_ref[...]

    def body1(g, carry):
        ssum, ssq = carry
        t = jnp.dot(x_ref[g].astype(jnp.bfloat16), a,
                    preferred_element_type=jnp.float32)
        bb = g // n

        @pl.when(g % n == 0)
        def _():
            ts_ref[bb] = t

        @pl.when(g % n != 0)
        def _():
            ts_ref[bb] = ts_ref[bb] + t

        z = jnp.dot(wt, t.astype(jnp.bfloat16),
                    preferred_element_type=jnp.float32) + b
        z_ref[g] = z
        return (ssum + jnp.sum(z, axis=1, keepdims=True),
                ssq + jnp.sum(z * z, axis=1, keepdims=True))

    zc = jnp.zeros((cout, 1), jnp.float32)
    ssum, ssq = lax.fori_loop(0, G, body1, (zc, zc))
    mean = ssum / (G * L)
    var = ssq / (G * L) - mean * mean
    rstd = lax.rsqrt(var + 1e-5)

    def body2(bb, carry):
        ssum, ssq = carry
        z2 = jnp.dot(wst, ts_ref[bb].astype(jnp.bfloat16),
                     preferred_element_type=jnp.float32) + bs
        zs_ref[bb] = z2
        return (ssum + jnp.sum(z2, axis=1, keepdims=True),
                ssq + jnp.sum(z2 * z2, axis=1, keepdims=True))

    s2sum, s2sq = lax.fori_loop(0, B, body2, (zc, zc))
    mean2 = s2sum / (B * L)
    var2 = s2sq / (B * L) - mean2 * mean2
    rstd2 = lax.rsqrt(var2 + 1e-5)

    sc1 = rstd * g_ref[...]
    of1 = be_ref[...] - mean * sc1
    sc2 = rstd2 * gs_ref[...]
    of2 = bes_ref[...] - mean2 * sc2

    def body3(g, c):
        z1 = z_ref[g] * sc1 + of1
        z2 = zs_ref[g // n] * sc2 + of2
        out_ref[g] = jnp.maximum(z1 + z2, 0.0).astype(out_dtype)
        return c

    lax.fori_loop(0, G, body3, 0)


def _stage(x, a, wt, b, gam, bet, wst, bs, gs, bes, out_dtype):
    cout = wt.shape[0]
    cin = wt.shape[1]
    G = x.shape[0]
    B = 4
    return pl.pallas_call(
        functools.partial(_stage_body, G, B, out_dtype),
        out_shape=jax.ShapeDtypeStruct((G, cout, L), out_dtype),
        scratch_shapes=[pltpu.VMEM((G, cout, L), jnp.float32),
                        pltpu.VMEM((B, cout, L), jnp.float32),
                        pltpu.VMEM((B, cin, L), jnp.float32)],
    )(x, a, wt, b, gam, bet, wst, bs, gs, bes)


def _col(v):
    return v.reshape(-1, 1)


def kernel(x, edge_index, W1, b1, g1, be1, W1s, b1s, g1s, be1s,
           W2, b2, g2, be2, W2s, b2s, g2s, be2s,
           W3, b3, g3, be3, W3s, b3s, g3s, be3s):
    ei = edge_index.astype(jnp.int32)
    counts2, degp = _sc_counts(ei[0], ei[1])
    a = _prep(counts2.reshape(NC, L, L), degp.reshape(NC, L, 1))

    h = x.reshape(16, x.shape[2], L)
    h = _stage(h, a, W1.T, _col(b1), _col(g1), _col(be1),
               W1s.T, _col(b1s), _col(g1s), _col(be1s), jnp.bfloat16)
    h = _stage(h, a, W2.T, _col(b2), _col(g2), _col(be2),
               W2s.T, _col(b2s), _col(g2s), _col(be2s), jnp.bfloat16)
    h = _stage(h, a, W3.T, _col(b3), _col(g3), _col(be3),
               W3s.T, _col(b3s), _col(g3s), _col(be3s), jnp.float32)
    return h


# speedup vs baseline: 1.1870x; 1.0953x over previous
import functools

import jax
import jax.numpy as jnp
from jax import lax
from jax.experimental import pallas as pl
from jax.experimental.pallas import tpu as pltpu
from jax.experimental.pallas import tpu_sc as plsc

L = 1024
E = 8192
NC = 2
NS = 16
EPW = E // (NC * NS)
WPS = (L * L) // NS
ZCH = 8192


def _sc_counts(src, dst):
    mesh = plsc.VectorSubcoreMesh(core_axis_name="c", subcore_axis_name="s")

    @functools.partial(
        pl.kernel,
        mesh=mesh,
        out_type=(jax.ShapeDtypeStruct((NC, L * L), jnp.float32),
                  jax.ShapeDtypeStruct((NC, L), jnp.float32)),
        scratch_types=[
            pltpu.VMEM((EPW,), jnp.int32),
            pltpu.VMEM((EPW // 128, 128), jnp.int32),
            pltpu.VMEM((EPW // 128, 128), jnp.int32),
            pltpu.VMEM((128,), jnp.float32),
            pltpu.VMEM((ZCH,), jnp.float32),
            pltpu.VMEM_SHARED((L * L,), jnp.float32),
            pltpu.VMEM_SHARED((L,), jnp.float32),
        ],
    )
    def k(src_hbm, dst_hbm, out_hbm, deg_hbm, sv, dv, iv, ones_v, zv, csh,
          dsh):
        cid = lax.axis_index("c")
        sid = lax.axis_index("s")

        def fill16(i, ref, val):
            ref[pl.ds(i * 16, 16)] = jnp.full((16,), val, ref.dtype)

        lax.fori_loop(0, ZCH // 16, lambda i, c: (fill16(i, zv, 0.0), c)[1], 0)
        lax.fori_loop(0, 128 // 16, lambda i, c: (fill16(i, ones_v, 1.0), c)[1], 0)

        base = sid * WPS

        def zc(j, c):
            pltpu.sync_copy(zv, csh.at[pl.ds(base + j * ZCH, ZCH)])
            return c

        lax.fori_loop(0, WPS // ZCH, zc, 0)

        @pl.when(sid == 0)
        def _():
            pltpu.sync_copy(zv.at[pl.ds(0, L)], dsh)

        plsc.subcore_barrier()

        ebase = (cid * NS + sid) * EPW
        pltpu.sync_copy(src_hbm.at[pl.ds(ebase, EPW)], sv)
        for g in range(EPW // 128):
            pltpu.sync_copy(dst_hbm.at[pl.ds(ebase + g * 128, 128)],
                            dv.at[g])
        for g in range(EPW // 128):
            for j in range(128 // 16):
                s16 = sv[pl.ds(g * 128 + j * 16, 16)]
                d16 = dv[g, pl.ds(j * 16, 16)]
                iv[g, pl.ds(j * 16, 16)] = s16 * L + d16

        for g in range(EPW // 128):
            pltpu.sync_copy(ones_v, csh.at[iv.at[g]], add=True)
            pltpu.sync_copy(ones_v, dsh.at[dv.at[g]], add=True)
        plsc.subcore_barrier()

        def co(j, c):
            pltpu.sync_copy(csh.at[pl.ds(base + j * ZCH, ZCH)],
                            out_hbm.at[cid, pl.ds(base + j * ZCH, ZCH)])
            return c

        lax.fori_loop(0, WPS // ZCH, co, 0)

        @pl.when(sid == 0)
        def _():
            pltpu.sync_copy(dsh, deg_hbm.at[cid])

    return k(src, dst)


def _prep_body(ct2_ref, degp_ref, a_ref):
    ct = ct2_ref[0] + ct2_ref[1]
    degc = degp_ref[0] + degp_ref[1] + 2.0
    dic = lax.rsqrt(degc)
    degr = jnp.sum(ct, axis=0, keepdims=True) + 2.0
    dir_ = lax.rsqrt(degr)
    eyem = jnp.where(
        lax.broadcasted_iota(jnp.int32, (L, L), 0)
        == lax.broadcasted_iota(jnp.int32, (L, L), 1),
        1.0, 0.0)
    a = dic * ct * dir_ + eyem * (2.0 * dir_ * dir_)
    a_ref[...] = a.astype(jnp.bfloat16)


def _prep(counts2, degp):
    return pl.pallas_call(
        _prep_body,
        out_shape=jax.ShapeDtypeStruct((L, L), jnp.bfloat16),
    )(counts2, degp)


def _one_stage(G, B, cin, a, read_in, mid_ref,
               wt_ref, b_ref, g_ref, be_ref,
               wst_ref, bs_ref, gs_ref, bes_ref, ts_ref, zs_ref):
    cout = wt_ref.shape[0]
    n = G // B
    wt = wt_ref[...].astype(jnp.bfloat16)
    wst = wst_ref[...].astype(jnp.bfloat16)
    b = b_ref[...]
    bs = bs_ref[...]

    def body1(g, carry):
        ssum, ssq = carry
        t = jnp.dot(read_in(g), a, preferred_element_type=jnp.float32)
        bb = g // n

        @pl.when(g % n == 0)
        def _():
            ts_ref[bb, pl.ds(0, cin), :] = t

        @pl.when(g % n != 0)
        def _():
            ts_ref[bb, pl.ds(0, cin), :] = ts_ref[bb, pl.ds(0, cin), :] + t

        z = jnp.dot(wt, t.astype(jnp.bfloat16),
                    preferred_element_type=jnp.float32) + b
        mid_ref[g] = z.astype(mid_ref.dtype)
        return (ssum + jnp.sum(z, axis=1, keepdims=True),
                ssq + jnp.sum(z * z, axis=1, keepdims=True))

    zc = jnp.zeros((cout, 1), jnp.float32)
    ssum, ssq = lax.fori_loop(0, G, body1, (zc, zc))
    mean = ssum / (G * L)
    var = ssq / (G * L) - mean * mean
    rstd = lax.rsqrt(var + 1e-5)

    def body2(bb, carry):
        ssum, ssq = carry
        t = ts_ref[bb, pl.ds(0, cin), :]
        z2 = jnp.dot(wst, t.astype(jnp.bfloat16),
                     preferred_element_type=jnp.float32) + bs
        zs_ref[bb] = z2
        return (ssum + jnp.sum(z2, axis=1, keepdims=True),
                ssq + jnp.sum(z2 * z2, axis=1, keepdims=True))

    s2sum, s2sq = lax.fori_loop(0, B, body2, (zc, zc))
    mean2 = s2sum / (B * L)
    var2 = s2sq / (B * L) - mean2 * mean2
    rstd2 = lax.rsqrt(var2 + 1e-5)

    sc1 = rstd * g_ref[...]
    of1 = be_ref[...] - mean * sc1
    sc2 = rstd2 * gs_ref[...]
    of2 = bes_ref[...] - mean2 * sc2

    def body3(g, c):
        z1 = mid_ref[g] * sc1 + of1
        z2 = zs_ref[g // n] * sc2 + of2
        mid_ref[g] = jnp.maximum(z1 + z2, 0.0).astype(mid_ref.dtype)
        return c

    lax.fori_loop(0, G, body3, 0)


def _fwd_body(G, B, x_ref, a_ref,
              wt1_ref, b1_ref, g1_ref, be1_ref,
              ws1_ref, bs1_ref, gs1_ref, bes1_ref,
              wt2_ref, b2_ref, g2_ref, be2_ref,
              ws2_ref, bs2_ref, gs2_ref, bes2_ref,
              wt3_ref, b3_ref, g3_ref, be3_ref,
              ws3_ref, bs3_ref, gs3_ref, bes3_ref,
              out_ref, ha_ref, ts_ref, zs_ref):
    a = a_ref[...]
    _one_stage(G, B, x_ref.shape[1], a,
               lambda g: x_ref[g].astype(jnp.bfloat16), ha_ref,
               wt1_ref, b1_ref, g1_ref, be1_ref,
               ws1_ref, bs1_ref, gs1_ref, bes1_ref, ts_ref, zs_ref)
    _one_stage(G, B, ha_ref.shape[1], a,
               lambda g: ha_ref[g], ha_ref,
               wt2_ref, b2_ref, g2_ref, be2_ref,
               ws2_ref, bs2_ref, gs2_ref, bes2_ref, ts_ref, zs_ref)
    _one_stage(G, B, ha_ref.shape[1], a,
               lambda g: ha_ref[g], out_ref,
               wt3_ref, b3_ref, g3_ref, be3_ref,
               ws3_ref, bs3_ref, gs3_ref, bes3_ref, ts_ref, zs_ref)


def _fwd(x, a, params):
    G = x.shape[0]
    B = 4
    cout = 256
    return pl.pallas_call(
        functools.partial(_fwd_body, G, B),
        out_shape=jax.ShapeDtypeStruct((G, cout, L), jnp.float32),
        scratch_shapes=[pltpu.VMEM((G, cout, L), jnp.bfloat16),
                        pltpu.VMEM((B, cout, L), jnp.float32),
                        pltpu.VMEM((B, cout, L), jnp.float32)],
    )(x, a, *params)


def _col(v):
    return v.reshape(-1, 1)


def kernel(x, edge_index, W1, b1, g1, be1, W1s, b1s, g1s, be1s,
           W2, b2, g2, be2, W2s, b2s, g2s, be2s,
           W3, b3, g3, be3, W3s, b3s, g3s, be3s):
    ei = edge_index.astype(jnp.int32)
    counts2, degp = _sc_counts(ei[0], ei[1])
    a = _prep(counts2.reshape(NC, L, L), degp.reshape(NC, L, 1))

    params = (W1.T, _col(b1), _col(g1), _col(be1),
              W1s.T, _col(b1s), _col(g1s), _col(be1s),
              W2.T, _col(b2), _col(g2), _col(be2),
              W2s.T, _col(b2s), _col(g2s), _col(be2s),
              W3.T, _col(b3), _col(g3), _col(be3),
              W3s.T, _col(b3s), _col(g3s), _col(be3s))
    return _fwd(x.reshape(16, x.shape[2], L), a, params)


# speedup vs baseline: 1.2666x; 1.0670x over previous
import functools

import jax
import jax.numpy as jnp
from jax import lax
from jax.experimental import pallas as pl
from jax.experimental.pallas import tpu as pltpu
from jax.experimental.pallas import tpu_sc as plsc

L = 1024
E = 8192
NC = 2
NS = 16
EPW = E // (NC * NS)
WPS = (L * L) // NS
ZCH = 8192


def _sc_counts(src, dst):
    mesh = plsc.VectorSubcoreMesh(core_axis_name="c", subcore_axis_name="s")

    @functools.partial(
        pl.kernel,
        mesh=mesh,
        out_type=(jax.ShapeDtypeStruct((NC, L * L), jnp.float32),
                  jax.ShapeDtypeStruct((NC, L), jnp.float32)),
        scratch_types=[
            pltpu.VMEM((EPW,), jnp.int32),
            pltpu.VMEM((EPW // 128, 128), jnp.int32),
            pltpu.VMEM((EPW // 128, 128), jnp.int32),
            pltpu.VMEM((128,), jnp.float32),
            pltpu.VMEM((ZCH,), jnp.float32),
            pltpu.VMEM_SHARED((L * L,), jnp.float32),
            pltpu.VMEM_SHARED((L,), jnp.float32),
        ],
    )
    def k(src_hbm, dst_hbm, out_hbm, deg_hbm, sv, dv, iv, ones_v, zv, csh,
          dsh):
        cid = lax.axis_index("c")
        sid = lax.axis_index("s")

        def fill16(i, ref, val):
            ref[pl.ds(i * 16, 16)] = jnp.full((16,), val, ref.dtype)

        lax.fori_loop(0, ZCH // 16, lambda i, c: (fill16(i, zv, 0.0), c)[1], 0)
        lax.fori_loop(0, 128 // 16, lambda i, c: (fill16(i, ones_v, 1.0), c)[1], 0)

        base = sid * WPS

        def zc(j, c):
            pltpu.sync_copy(zv, csh.at[pl.ds(base + j * ZCH, ZCH)])
            return c

        lax.fori_loop(0, WPS // ZCH, zc, 0)

        @pl.when(sid == 0)
        def _():
            pltpu.sync_copy(zv.at[pl.ds(0, L)], dsh)

        plsc.subcore_barrier()

        ebase = (cid * NS + sid) * EPW
        pltpu.sync_copy(src_hbm.at[pl.ds(ebase, EPW)], sv)
        for g in range(EPW // 128):
            pltpu.sync_copy(dst_hbm.at[pl.ds(ebase + g * 128, 128)],
                            dv.at[g])
        for g in range(EPW // 128):
            for j in range(128 // 16):
                s16 = sv[pl.ds(g * 128 + j * 16, 16)]
                d16 = dv[g, pl.ds(j * 16, 16)]
                iv[g, pl.ds(j * 16, 16)] = s16 * L + d16

        for g in range(EPW // 128):
            pltpu.sync_copy(ones_v, csh.at[iv.at[g]], add=True)
            pltpu.sync_copy(ones_v, dsh.at[dv.at[g]], add=True)
        plsc.subcore_barrier()

        def co(j, c):
            pltpu.sync_copy(csh.at[pl.ds(base + j * ZCH, ZCH)],
                            out_hbm.at[cid, pl.ds(base + j * ZCH, ZCH)])
            return c

        lax.fori_loop(0, WPS // ZCH, co, 0)

        @pl.when(sid == 0)
        def _():
            pltpu.sync_copy(dsh, deg_hbm.at[cid])

    return k(src, dst)


def _prep_body(ct2_ref, degp_ref, a_ref):
    ct = ct2_ref[0] + ct2_ref[1]
    degc = degp_ref[0] + degp_ref[1] + 2.0
    dic = lax.rsqrt(degc)
    degr = jnp.sum(ct, axis=0, keepdims=True) + 2.0
    dir_ = lax.rsqrt(degr)
    eyem = jnp.where(
        lax.broadcasted_iota(jnp.int32, (L, L), 0)
        == lax.broadcasted_iota(jnp.int32, (L, L), 1),
        1.0, 0.0)
    a = dic * ct * dir_ + eyem * (2.0 * dir_ * dir_)
    a_ref[...] = a.astype(jnp.bfloat16)


def _prep(counts2, degp):
    return pl.pallas_call(
        _prep_body,
        out_shape=jax.ShapeDtypeStruct((L, L), jnp.bfloat16),
    )(counts2, degp)


def _one_stage(G, B, cin, a, read_all, mid_ref,
               wt_ref, b_ref, g_ref, be_ref,
               wst_ref, bs_ref, gs_ref, bes_ref, t_ref, zs_ref):
    cout = wt_ref.shape[0]
    n = G // B
    wt = wt_ref[...].astype(jnp.bfloat16)
    wst = wst_ref[...].astype(jnp.bfloat16)
    b = b_ref[...]
    bs = bs_ref[...]

    H = G // 4
    for c in range(4):
        xh = read_all(c * H, H)
        th = jnp.dot(xh, a, preferred_element_type=jnp.float32)
        t_ref[pl.ds(c * H, H), pl.ds(0, cin), :] = (
            th.astype(jnp.bfloat16).reshape(H, cin, L))

    def body1(g, carry):
        ssum, ssq = carry
        z = jnp.dot(wt, t_ref[g, pl.ds(0, cin), :],
                    preferred_element_type=jnp.float32) + b
        mid_ref[g] = z.astype(mid_ref.dtype)
        return (ssum + jnp.sum(z, axis=1, keepdims=True),
                ssq + jnp.sum(z * z, axis=1, keepdims=True))

    zc = jnp.zeros((cout, 1), jnp.float32)
    ssum, ssq = lax.fori_loop(0, G, body1, (zc, zc))
    mean = ssum / (G * L)
    var = ssq / (G * L) - mean * mean
    rstd = lax.rsqrt(var + 1e-5)

    def body2(bb, carry):
        ssum, ssq = carry
        ts = t_ref[n * bb, pl.ds(0, cin), :].astype(jnp.float32)
        for i in range(1, n):
            ts = ts + t_ref[n * bb + i, pl.ds(0, cin), :].astype(jnp.float32)
        z2 = jnp.dot(wst, ts.astype(jnp.bfloat16),
                     preferred_element_type=jnp.float32) + bs
        zs_ref[bb] = z2
        return (ssum + jnp.sum(z2, axis=1, keepdims=True),
                ssq + jnp.sum(z2 * z2, axis=1, keepdims=True))

    s2sum, s2sq = lax.fori_loop(0, B, body2, (zc, zc))
    mean2 = s2sum / (B * L)
    var2 = s2sq / (B * L) - mean2 * mean2
    rstd2 = lax.rsqrt(var2 + 1e-5)

    sc1 = rstd * g_ref[...]
    of1 = be_ref[...] - mean * sc1
    sc2 = rstd2 * gs_ref[...]
    of2 = bes_ref[...] - mean2 * sc2

    def body3(g, c):
        z1 = mid_ref[g] * sc1 + of1
        z2 = zs_ref[g // n] * sc2 + of2
        mid_ref[g] = jnp.maximum(z1 + z2, 0.0).astype(mid_ref.dtype)
        return c

    lax.fori_loop(0, G, body3, 0)


def _fwd_body(G, B, x_ref, a_ref,
              wt1_ref, b1_ref, g1_ref, be1_ref,
              ws1_ref, bs1_ref, gs1_ref, bes1_ref,
              wt2_ref, b2_ref, g2_ref, be2_ref,
              ws2_ref, bs2_ref, gs2_ref, bes2_ref,
              wt3_ref, b3_ref, g3_ref, be3_ref,
              ws3_ref, bs3_ref, gs3_ref, bes3_ref,
              out_ref, ha_ref, t_ref, zs_ref):
    a = a_ref[...]
    cin1 = x_ref.shape[1]
    cmid = ha_ref.shape[1]

    def rd_x(lo, m):
        return (x_ref[pl.ds(lo, m)].astype(jnp.bfloat16)
                .reshape(m * cin1, L))

    def rd_h(lo, m):
        return ha_ref[pl.ds(lo, m)].reshape(m * cmid, L)

    _one_stage(G, B, cin1, a, rd_x, ha_ref,
               wt1_ref, b1_ref, g1_ref, be1_ref,
               ws1_ref, bs1_ref, gs1_ref, bes1_ref, t_ref, zs_ref)
    _one_stage(G, B, cmid, a, rd_h, ha_ref,
               wt2_ref, b2_ref, g2_ref, be2_ref,
               ws2_ref, bs2_ref, gs2_ref, bes2_ref, t_ref, zs_ref)
    _one_stage(G, B, cmid, a, rd_h, out_ref,
               wt3_ref, b3_ref, g3_ref, be3_ref,
               ws3_ref, bs3_ref, gs3_ref, bes3_ref, t_ref, zs_ref)


def _fwd(x, a, params):
    G = x.shape[0]
    B = 4
    cout = 256
    return pl.pallas_call(
        functools.partial(_fwd_body, G, B),
        out_shape=jax.ShapeDtypeStruct((G, cout, L), jnp.float32),
        scratch_shapes=[pltpu.VMEM((G, cout, L), jnp.bfloat16),
                        pltpu.VMEM((G, cout, L), jnp.bfloat16),
                        pltpu.VMEM((B, cout, L), jnp.float32)],
    )(x, a, *params)


def _col(v):
    return v.reshape(-1, 1)


def kernel(x, edge_index, W1, b1, g1, be1, W1s, b1s, g1s, be1s,
           W2, b2, g2, be2, W2s, b2s, g2s, be2s,
           W3, b3, g3, be3, W3s, b3s, g3s, be3s):
    ei = edge_index.astype(jnp.int32)
    counts2, degp = _sc_counts(ei[0], ei[1])
    a = _prep(counts2.reshape(NC, L, L), degp.reshape(NC, L, 1))

    params = (W1.T, _col(b1), _col(g1), _col(be1),
              W1s.T, _col(b1s), _col(g1s), _col(be1s),
              W2.T, _col(b2), _col(g2), _col(be2),
              W2s.T, _col(b2s), _col(g2s), _col(be2s),
              W3.T, _col(b3), _col(g3), _col(be3),
              W3s.T, _col(b3s), _col(g3s), _col(be3s))
    return _fwd(x.reshape(16, x.shape[2], L), a, params)


# speedup vs baseline: 1.4441x; 1.1402x over previous
import functools

import jax
import jax.numpy as jnp
from jax import lax
from jax.experimental import pallas as pl
from jax.experimental.pallas import tpu as pltpu
from jax.experimental.pallas import tpu_sc as plsc

L = 1024
E = 8192
NC = 2
NS = 16
EPW = E // (NC * NS)
WPS = (L * L) // NS
ZCH = 8192


def _sc_counts(src, dst):
    mesh = plsc.VectorSubcoreMesh(core_axis_name="c", subcore_axis_name="s")

    @functools.partial(
        pl.kernel,
        mesh=mesh,
        out_type=(jax.ShapeDtypeStruct((NC, L * L), jnp.float32),
                  jax.ShapeDtypeStruct((NC, L), jnp.float32)),
        scratch_types=[
            pltpu.VMEM((EPW,), jnp.int32),
            pltpu.VMEM((EPW // 128, 128), jnp.int32),
            pltpu.VMEM((EPW // 128, 128), jnp.int32),
            pltpu.VMEM((128,), jnp.float32),
            pltpu.VMEM((ZCH,), jnp.float32),
            pltpu.VMEM_SHARED((L * L,), jnp.float32),
            pltpu.VMEM_SHARED((L,), jnp.float32),
        ],
    )
    def k(src_hbm, dst_hbm, out_hbm, deg_hbm, sv, dv, iv, ones_v, zv, csh,
          dsh):
        cid = lax.axis_index("c")
        sid = lax.axis_index("s")

        def fill16(i, ref, val):
            ref[pl.ds(i * 16, 16)] = jnp.full((16,), val, ref.dtype)

        lax.fori_loop(0, ZCH // 16, lambda i, c: (fill16(i, zv, 0.0), c)[1], 0)
        lax.fori_loop(0, 128 // 16, lambda i, c: (fill16(i, ones_v, 1.0), c)[1], 0)

        base = sid * WPS

        def zc(j, c):
            pltpu.sync_copy(zv, csh.at[pl.ds(base + j * ZCH, ZCH)])
            return c

        lax.fori_loop(0, WPS // ZCH, zc, 0)

        @pl.when(sid == 0)
        def _():
            pltpu.sync_copy(zv.at[pl.ds(0, L)], dsh)

        plsc.subcore_barrier()

        ebase = (cid * NS + sid) * EPW
        pltpu.sync_copy(src_hbm.at[pl.ds(ebase, EPW)], sv)
        for g in range(EPW // 128):
            pltpu.sync_copy(dst_hbm.at[pl.ds(ebase + g * 128, 128)],
                            dv.at[g])
        for g in range(EPW // 128):
            for j in range(128 // 16):
                s16 = sv[pl.ds(g * 128 + j * 16, 16)]
                d16 = dv[g, pl.ds(j * 16, 16)]
                iv[g, pl.ds(j * 16, 16)] = s16 * L + d16

        for g in range(EPW // 128):
            pltpu.sync_copy(ones_v, csh.at[iv.at[g]], add=True)
            pltpu.sync_copy(ones_v, dsh.at[dv.at[g]], add=True)
        plsc.subcore_barrier()

        def co(j, c):
            pltpu.sync_copy(csh.at[pl.ds(base + j * ZCH, ZCH)],
                            out_hbm.at[cid, pl.ds(base + j * ZCH, ZCH)])
            return c

        lax.fori_loop(0, WPS // ZCH, co, 0)

        @pl.when(sid == 0)
        def _():
            pltpu.sync_copy(dsh, deg_hbm.at[cid])

    return k(src, dst)


def _prep_body(ct2_ref, degp_ref, a_ref):
    ct = ct2_ref[0] + ct2_ref[1]
    degc = degp_ref[0] + degp_ref[1] + 2.0
    dic = lax.rsqrt(degc)
    degr = jnp.sum(ct, axis=0, keepdims=True) + 2.0
    dir_ = lax.rsqrt(degr)
    eyem = jnp.where(
        lax.broadcasted_iota(jnp.int32, (L, L), 0)
        == lax.broadcasted_iota(jnp.int32, (L, L), 1),
        1.0, 0.0)
    a = dic * ct * dir_ + eyem * (2.0 * dir_ * dir_)
    a_ref[...] = a.astype(jnp.bfloat16)


def _prep(counts2, degp):
    return pl.pallas_call(
        _prep_body,
        out_shape=jax.ShapeDtypeStruct((L, L), jnp.bfloat16),
    )(counts2, degp)


def _one_stage(G, B, cin, a, read_all, mid_ref,
               wt_ref, b_ref, g_ref, be_ref,
               wst_ref, bs_ref, gs_ref, bes_ref, t_ref, zs_ref):
    cout = wt_ref.shape[0]
    n = G // B
    wt = wt_ref[...].astype(jnp.bfloat16)
    wst = wst_ref[...].astype(jnp.bfloat16)
    b = b_ref[...]
    bs = bs_ref[...]

    H = G // 4
    for c in range(4):
        xh = read_all(c * H, H)
        th = jnp.dot(xh, a, preferred_element_type=jnp.float32)
        t_ref[pl.ds(c * H, H), pl.ds(0, cin), :] = (
            th.astype(jnp.bfloat16).reshape(H, cin, L))

    def body1(bb, carry):
        ssum, ssq = carry
        tc = jnp.concatenate(
            [t_ref[n * bb + i, pl.ds(0, cin), :] for i in range(n)], axis=1)
        z = jnp.dot(wt, tc, preferred_element_type=jnp.float32) + b
        for i in range(n):
            mid_ref[n * bb + i] = z[:, i * L:(i + 1) * L].astype(
                mid_ref.dtype)
        return (ssum + jnp.sum(z, axis=1, keepdims=True),
                ssq + jnp.sum(z * z, axis=1, keepdims=True))

    zc = jnp.zeros((cout, 1), jnp.float32)
    ssum, ssq = lax.fori_loop(0, B, body1, (zc, zc))
    mean = ssum / (G * L)
    var = ssq / (G * L) - mean * mean
    rstd = lax.rsqrt(var + 1e-5)

    def body2(bb, carry):
        ssum, ssq = carry
        ts = t_ref[n * bb, pl.ds(0, cin), :].astype(jnp.float32)
        for i in range(1, n):
            ts = ts + t_ref[n * bb + i, pl.ds(0, cin), :].astype(jnp.float32)
        z2 = jnp.dot(wst, ts.astype(jnp.bfloat16),
                     preferred_element_type=jnp.float32) + bs
        zs_ref[bb] = z2
        return (ssum + jnp.sum(z2, axis=1, keepdims=True),
                ssq + jnp.sum(z2 * z2, axis=1, keepdims=True))

    s2sum, s2sq = lax.fori_loop(0, B, body2, (zc, zc))
    mean2 = s2sum / (B * L)
    var2 = s2sq / (B * L) - mean2 * mean2
    rstd2 = lax.rsqrt(var2 + 1e-5)

    sc1 = rstd * g_ref[...]
    of1 = be_ref[...] - mean * sc1
    sc2 = rstd2 * gs_ref[...]
    of2 = bes_ref[...] - mean2 * sc2

    def body3(bb, c):
        z2 = zs_ref[bb] * sc2 + of2
        for i in range(n):
            g = n * bb + i
            mid_ref[g] = jnp.maximum(mid_ref[g] * sc1 + of1 + z2,
                                     0.0).astype(mid_ref.dtype)
        return c

    lax.fori_loop(0, B, body3, 0)


def _fwd_body(G, B, x_ref, a_ref,
              wt1_ref, b1_ref, g1_ref, be1_ref,
              ws1_ref, bs1_ref, gs1_ref, bes1_ref,
              wt2_ref, b2_ref, g2_ref, be2_ref,
              ws2_ref, bs2_ref, gs2_ref, bes2_ref,
              wt3_ref, b3_ref, g3_ref, be3_ref,
              ws3_ref, bs3_ref, gs3_ref, bes3_ref,
              out_ref, ha_ref, t_ref, zs_ref):
    a = a_ref[...]
    cin1 = x_ref.shape[1]
    cmid = ha_ref.shape[1]

    def rd_x(lo, m):
        return (x_ref[pl.ds(lo, m)].astype(jnp.bfloat16)
                .reshape(m * cin1, L))

    def rd_h(lo, m):
        return ha_ref[pl.ds(lo, m)].reshape(m * cmid, L)

    _one_stage(G, B, cin1, a, rd_x, ha_ref,
               wt1_ref, b1_ref, g1_ref, be1_ref,
               ws1_ref, bs1_ref, gs1_ref, bes1_ref, t_ref, zs_ref)
    _one_stage(G, B, cmid, a, rd_h, ha_ref,
               wt2_ref, b2_ref, g2_ref, be2_ref,
               ws2_ref, bs2_ref, gs2_ref, bes2_ref, t_ref, zs_ref)
    _one_stage(G, B, cmid, a, rd_h, out_ref,
               wt3_ref, b3_ref, g3_ref, be3_ref,
               ws3_ref, bs3_ref, gs3_ref, bes3_ref, t_ref, zs_ref)


def _fwd(x, a, params):
    G = x.shape[0]
    B = 4
    cout = 256
    return pl.pallas_call(
        functools.partial(_fwd_body, G, B),
        out_shape=jax.ShapeDtypeStruct((G, cout, L), jnp.float32),
        scratch_shapes=[pltpu.VMEM((G, cout, L), jnp.bfloat16),
                        pltpu.VMEM((G, cout, L), jnp.bfloat16),
                        pltpu.VMEM((B, cout, L), jnp.float32)],
    )(x, a, *params)


def _col(v):
    return v.reshape(-1, 1)


def kernel(x, edge_index, W1, b1, g1, be1, W1s, b1s, g1s, be1s,
           W2, b2, g2, be2, W2s, b2s, g2s, be2s,
           W3, b3, g3, be3, W3s, b3s, g3s, be3s):
    ei = edge_index.astype(jnp.int32)
    counts2, degp = _sc_counts(ei[0], ei[1])
    a = _prep(counts2.reshape(NC, L, L), degp.reshape(NC, L, 1))

    params = (W1.T, _col(b1), _col(g1), _col(be1),
              W1s.T, _col(b1s), _col(g1s), _col(be1s),
              W2.T, _col(b2), _col(g2), _col(be2),
              W2s.T, _col(b2s), _col(g2s), _col(be2s),
              W3.T, _col(b3), _col(g3), _col(be3),
              W3s.T, _col(b3s), _col(g3s), _col(be3s))
    return _fwd(x.reshape(16, x.shape[2], L), a, params)


# speedup vs baseline: 1.4934x; 1.0341x over previous
import functools

import jax
import jax.numpy as jnp
from jax import lax
from jax.experimental import pallas as pl
from jax.experimental.pallas import tpu as pltpu
from jax.experimental.pallas import tpu_sc as plsc

L = 1024
E = 8192
NC = 2
NS = 16
EPW = E // (NC * NS)
WPS = (L * L) // NS
ZCH = 8192


def _sc_counts(src, dst):
    mesh = plsc.VectorSubcoreMesh(core_axis_name="c", subcore_axis_name="s")

    @functools.partial(
        pl.kernel,
        mesh=mesh,
        out_type=(jax.ShapeDtypeStruct((NC, L * L), jnp.float32),
                  jax.ShapeDtypeStruct((NC, L), jnp.float32)),
        scratch_types=[
            pltpu.VMEM((EPW,), jnp.int32),
            pltpu.VMEM((EPW // 128, 128), jnp.int32),
            pltpu.VMEM((EPW // 128, 128), jnp.int32),
            pltpu.VMEM((128,), jnp.float32),
            pltpu.VMEM((ZCH,), jnp.float32),
            pltpu.VMEM_SHARED((L * L,), jnp.float32),
            pltpu.VMEM_SHARED((L,), jnp.float32),
        ],
    )
    def k(src_hbm, dst_hbm, out_hbm, deg_hbm, sv, dv, iv, ones_v, zv, csh,
          dsh):
        cid = lax.axis_index("c")
        sid = lax.axis_index("s")

        def fill16(i, ref, val):
            ref[pl.ds(i * 16, 16)] = jnp.full((16,), val, ref.dtype)

        lax.fori_loop(0, ZCH // 16, lambda i, c: (fill16(i, zv, 0.0), c)[1], 0)
        lax.fori_loop(0, 128 // 16, lambda i, c: (fill16(i, ones_v, 1.0), c)[1], 0)

        base = sid * WPS

        def zc(j, c):
            pltpu.sync_copy(zv, csh.at[pl.ds(base + j * ZCH, ZCH)])
            return c

        lax.fori_loop(0, WPS // ZCH, zc, 0)

        @pl.when(sid == 0)
        def _():
            pltpu.sync_copy(zv.at[pl.ds(0, L)], dsh)

        plsc.subcore_barrier()

        ebase = (cid * NS + sid) * EPW
        pltpu.sync_copy(src_hbm.at[pl.ds(ebase, EPW)], sv)
        for g in range(EPW // 128):
            pltpu.sync_copy(dst_hbm.at[pl.ds(ebase + g * 128, 128)],
                            dv.at[g])
        for g in range(EPW // 128):
            for j in range(128 // 16):
                s16 = sv[pl.ds(g * 128 + j * 16, 16)]
                d16 = dv[g, pl.ds(j * 16, 16)]
                iv[g, pl.ds(j * 16, 16)] = s16 * L + d16

        for g in range(EPW // 128):
            pltpu.sync_copy(ones_v, csh.at[iv.at[g]], add=True)
            pltpu.sync_copy(ones_v, dsh.at[dv.at[g]], add=True)
        plsc.subcore_barrier()

        def co(j, c):
            pltpu.sync_copy(csh.at[pl.ds(base + j * ZCH, ZCH)],
                            out_hbm.at[cid, pl.ds(base + j * ZCH, ZCH)])
            return c

        lax.fori_loop(0, WPS // ZCH, co, 0)

        @pl.when(sid == 0)
        def _():
            pltpu.sync_copy(dsh, deg_hbm.at[cid])

    return k(src, dst)


def _prep_body(ct2_ref, degp_ref, a_ref):
    ct = ct2_ref[0] + ct2_ref[1]
    degc = degp_ref[0] + degp_ref[1] + 2.0
    dic = lax.rsqrt(degc)
    degr = jnp.sum(ct, axis=0, keepdims=True) + 2.0
    dir_ = lax.rsqrt(degr)
    eyem = jnp.where(
        lax.broadcasted_iota(jnp.int32, (L, L), 0)
        == lax.broadcasted_iota(jnp.int32, (L, L), 1),
        1.0, 0.0)
    a = dic * ct * dir_ + eyem * (2.0 * dir_ * dir_)
    a_ref[...] = a.astype(jnp.bfloat16)


def _prep(counts2, degp):
    return pl.pallas_call(
        _prep_body,
        out_shape=jax.ShapeDtypeStruct((L, L), jnp.bfloat16),
    )(counts2, degp)


def _one_stage(G, B, cin, a, read_all, mid_ref,
               wt_ref, b_ref, g_ref, be_ref,
               wst_ref, bs_ref, gs_ref, bes_ref, zs_ref):
    cout = wt_ref.shape[0]
    n = G // B
    wt = wt_ref[...].astype(jnp.bfloat16)
    wst = wst_ref[...].astype(jnp.bfloat16)
    b = b_ref[...]
    bs = bs_ref[...]

    def body1(bb, carry):
        ssum, ssq, s2sum, s2sq = carry
        xh = read_all(n * bb, n)
        th = jnp.dot(xh, a, preferred_element_type=jnp.float32)
        ts = th[0 * cin:1 * cin, :]
        for i in range(1, n):
            ts = ts + th[i * cin:(i + 1) * cin, :]
        z2 = jnp.dot(wst, ts.astype(jnp.bfloat16),
                     preferred_element_type=jnp.float32) + bs
        zs_ref[bb] = z2
        tb = th.astype(jnp.bfloat16)
        tc = jnp.concatenate(
            [tb[i * cin:(i + 1) * cin, :] for i in range(n)], axis=1)
        z = jnp.dot(wt, tc, preferred_element_type=jnp.float32) + b
        for i in range(n):
            mid_ref[n * bb + i] = z[:, i * L:(i + 1) * L].astype(
                mid_ref.dtype)
        return (ssum + jnp.sum(z, axis=1, keepdims=True),
                ssq + jnp.sum(z * z, axis=1, keepdims=True),
                s2sum + jnp.sum(z2, axis=1, keepdims=True),
                s2sq + jnp.sum(z2 * z2, axis=1, keepdims=True))

    zc = jnp.zeros((cout, 1), jnp.float32)
    ssum, ssq, s2sum, s2sq = lax.fori_loop(0, B, body1, (zc, zc, zc, zc))
    mean = ssum / (G * L)
    var = ssq / (G * L) - mean * mean
    rstd = lax.rsqrt(var + 1e-5)
    mean2 = s2sum / (B * L)
    var2 = s2sq / (B * L) - mean2 * mean2
    rstd2 = lax.rsqrt(var2 + 1e-5)

    sc1 = rstd * g_ref[...]
    of1 = be_ref[...] - mean * sc1
    sc2 = rstd2 * gs_ref[...]
    of2 = bes_ref[...] - mean2 * sc2

    def body3(bb, c):
        z2 = zs_ref[bb] * sc2 + of2
        for i in range(n):
            g = n * bb + i
            mid_ref[g] = jnp.maximum(mid_ref[g] * sc1 + of1 + z2,
                                     0.0).astype(mid_ref.dtype)
        return c

    lax.fori_loop(0, B, body3, 0)


def _fwd_body(G, B, x_ref, a_ref,
              wt1_ref, b1_ref, g1_ref, be1_ref,
              ws1_ref, bs1_ref, gs1_ref, bes1_ref,
              wt2_ref, b2_ref, g2_ref, be2_ref,
              ws2_ref, bs2_ref, gs2_ref, bes2_ref,
              wt3_ref, b3_ref, g3_ref, be3_ref,
              ws3_ref, bs3_ref, gs3_ref, bes3_ref,
              out_ref, ha_ref, zs_ref):
    a = a_ref[...]
    cin1 = x_ref.shape[1]
    cmid = ha_ref.shape[1]

    def rd_x(lo, m):
        return (x_ref[pl.ds(lo, m)].astype(jnp.bfloat16)
                .reshape(m * cin1, L))

    def rd_h(lo, m):
        return ha_ref[pl.ds(lo, m)].reshape(m * cmid, L)

    _one_stage(G, B, cin1, a, rd_x, ha_ref,
               wt1_ref, b1_ref, g1_ref, be1_ref,
               ws1_ref, bs1_ref, gs1_ref, bes1_ref, zs_ref)
    _one_stage(G, B, cmid, a, rd_h, ha_ref,
               wt2_ref, b2_ref, g2_ref, be2_ref,
               ws2_ref, bs2_ref, gs2_ref, bes2_ref, zs_ref)
    _one_stage(G, B, cmid, a, rd_h, out_ref,
               wt3_ref, b3_ref, g3_ref, be3_ref,
               ws3_ref, bs3_ref, gs3_ref, bes3_ref, zs_ref)


def _fwd(x, a, params):
    G = x.shape[0]
    B = 4
    cout = 256
    return pl.pallas_call(
        functools.partial(_fwd_body, G, B),
        out_shape=jax.ShapeDtypeStruct((G, cout, L), jnp.float32),
        scratch_shapes=[pltpu.VMEM((G, cout, L), jnp.bfloat16),
                        pltpu.VMEM((B, cout, L), jnp.float32)],
    )(x, a, *params)


def _col(v):
    return v.reshape(-1, 1)


def kernel(x, edge_index, W1, b1, g1, be1, W1s, b1s, g1s, be1s,
           W2, b2, g2, be2, W2s, b2s, g2s, be2s,
           W3, b3, g3, be3, W3s, b3s, g3s, be3s):
    ei = edge_index.astype(jnp.int32)
    counts2, degp = _sc_counts(ei[0], ei[1])
    a = _prep(counts2.reshape(NC, L, L), degp.reshape(NC, L, 1))

    params = (W1.T, _col(b1), _col(g1), _col(be1),
              W1s.T, _col(b1s), _col(g1s), _col(be1s),
              W2.T, _col(b2), _col(g2), _col(be2),
              W2s.T, _col(b2s), _col(g2s), _col(be2s),
              W3.T, _col(b3), _col(g3), _col(be3),
              W3s.T, _col(b3s), _col(g3s), _col(be3s))
    return _fwd(x.reshape(16, x.shape[2], L), a, params)


# speedup vs baseline: 1.5361x; 1.0286x over previous
import functools

import jax
import jax.numpy as jnp
from jax import lax
from jax.experimental import pallas as pl
from jax.experimental.pallas import tpu as pltpu
from jax.experimental.pallas import tpu_sc as plsc

L = 1024
E = 8192
NC = 2
NS = 16
EPW = E // (NC * NS)
WPS = (L * L) // NS
ZCH = 8192


def _sc_counts(src, dst):
    mesh = plsc.VectorSubcoreMesh(core_axis_name="c", subcore_axis_name="s")

    @functools.partial(
        pl.kernel,
        mesh=mesh,
        out_type=(jax.ShapeDtypeStruct((NC, L * L), jnp.float32),
                  jax.ShapeDtypeStruct((NC, L), jnp.float32)),
        scratch_types=[
            pltpu.VMEM((EPW,), jnp.int32),
            pltpu.VMEM((EPW // 128, 128), jnp.int32),
            pltpu.VMEM((EPW // 128, 128), jnp.int32),
            pltpu.VMEM((128,), jnp.float32),
            pltpu.VMEM((ZCH,), jnp.float32),
            pltpu.VMEM_SHARED((L * L,), jnp.float32),
            pltpu.VMEM_SHARED((L,), jnp.float32),
        ],
    )
    def k(src_hbm, dst_hbm, out_hbm, deg_hbm, sv, dv, iv, ones_v, zv, csh,
          dsh):
        cid = lax.axis_index("c")
        sid = lax.axis_index("s")

        def fill16(i, ref, val):
            ref[pl.ds(i * 16, 16)] = jnp.full((16,), val, ref.dtype)

        lax.fori_loop(0, ZCH // 16, lambda i, c: (fill16(i, zv, 0.0), c)[1], 0)
        lax.fori_loop(0, 128 // 16, lambda i, c: (fill16(i, ones_v, 1.0), c)[1], 0)

        base = sid * WPS

        def zc(j, c):
            pltpu.sync_copy(zv, csh.at[pl.ds(base + j * ZCH, ZCH)])
            return c

        lax.fori_loop(0, WPS // ZCH, zc, 0)

        @pl.when(sid == 0)
        def _():
            pltpu.sync_copy(zv.at[pl.ds(0, L)], dsh)

        plsc.subcore_barrier()

        ebase = (cid * NS + sid) * EPW
        pltpu.sync_copy(src_hbm.at[pl.ds(ebase, EPW)], sv)
        for g in range(EPW // 128):
            pltpu.sync_copy(dst_hbm.at[pl.ds(ebase + g * 128, 128)],
                            dv.at[g])
        for g in range(EPW // 128):
            for j in range(128 // 16):
                s16 = sv[pl.ds(g * 128 + j * 16, 16)]
                d16 = dv[g, pl.ds(j * 16, 16)]
                iv[g, pl.ds(j * 16, 16)] = s16 * L + d16

        for g in range(EPW // 128):
            pltpu.sync_copy(ones_v, csh.at[iv.at[g]], add=True)
            pltpu.sync_copy(ones_v, dsh.at[dv.at[g]], add=True)
        plsc.subcore_barrier()

        def co(j, c):
            pltpu.sync_copy(csh.at[pl.ds(base + j * ZCH, ZCH)],
                            out_hbm.at[cid, pl.ds(base + j * ZCH, ZCH)])
            return c

        lax.fori_loop(0, WPS // ZCH, co, 0)

        @pl.when(sid == 0)
        def _():
            pltpu.sync_copy(dsh, deg_hbm.at[cid])

    return k(src, dst)


def _prep_body(ct2_ref, degp_ref, a_ref):
    ct = ct2_ref[0] + ct2_ref[1]
    degc = degp_ref[0] + degp_ref[1] + 2.0
    dic = lax.rsqrt(degc)
    degr = jnp.sum(ct, axis=0, keepdims=True) + 2.0
    dir_ = lax.rsqrt(degr)
    eyem = jnp.where(
        lax.broadcasted_iota(jnp.int32, (L, L), 0)
        == lax.broadcasted_iota(jnp.int32, (L, L), 1),
        1.0, 0.0)
    a = dic * ct * dir_ + eyem * (2.0 * dir_ * dir_)
    a_ref[...] = a.astype(jnp.bfloat16)


def _prep(counts2, degp):
    return pl.pallas_call(
        _prep_body,
        out_shape=jax.ShapeDtypeStruct((L, L), jnp.bfloat16),
    )(counts2, degp)


def _one_stage(G, B, cin, a, read_all, mid_ref,
               wt_ref, b_ref, g_ref, be_ref,
               wst_ref, bs_ref, gs_ref, bes_ref, zs_ref):
    cout = wt_ref.shape[0]
    n = G // B
    wt = wt_ref[...].astype(jnp.bfloat16)
    wst = wst_ref[...].astype(jnp.bfloat16)
    b = b_ref[...]
    bs = bs_ref[...]

    def body1(bb, carry):
        ssum, ssq, s2sum, s2sq = carry
        xh = read_all(n * bb, n)
        th = jnp.dot(xh, a, preferred_element_type=jnp.float32)
        ts = th[0 * cin:1 * cin, :]
        for i in range(1, n):
            ts = ts + th[i * cin:(i + 1) * cin, :]
        z2 = jnp.dot(wst, ts.astype(jnp.bfloat16),
                     preferred_element_type=jnp.float32) + bs
        zs_ref[bb] = z2
        tb = th.astype(jnp.bfloat16)
        tc = jnp.concatenate(
            [tb[i * cin:(i + 1) * cin, :] for i in range(n)], axis=1)
        z = jnp.dot(wt, tc, preferred_element_type=jnp.float32) + b
        for i in range(n):
            mid_ref[n * bb + i] = z[:, i * L:(i + 1) * L].astype(
                mid_ref.dtype)
        return (ssum + jnp.sum(z, axis=1, keepdims=True),
                ssq + jnp.sum(z * z, axis=1, keepdims=True),
                s2sum + jnp.sum(z2, axis=1, keepdims=True),
                s2sq + jnp.sum(z2 * z2, axis=1, keepdims=True))

    zc = jnp.zeros((cout, 1), jnp.float32)
    carry = (zc, zc, zc, zc)
    for bb in range(B):
        carry = body1(bb, carry)
    ssum, ssq, s2sum, s2sq = carry
    mean = ssum / (G * L)
    var = ssq / (G * L) - mean * mean
    rstd = lax.rsqrt(var + 1e-5)
    mean2 = s2sum / (B * L)
    var2 = s2sq / (B * L) - mean2 * mean2
    rstd2 = lax.rsqrt(var2 + 1e-5)

    sc1 = rstd * g_ref[...]
    of1 = be_ref[...] - mean * sc1
    sc2 = rstd2 * gs_ref[...]
    of2 = bes_ref[...] - mean2 * sc2

    def body3(bb, c):
        z2 = zs_ref[bb] * sc2 + of2
        for i in range(n):
            g = n * bb + i
            mid_ref[g] = jnp.maximum(mid_ref[g] * sc1 + of1 + z2,
                                     0.0).astype(mid_ref.dtype)
        return c

    lax.fori_loop(0, B, body3, 0)


def _fwd_body(G, B, x_ref, a_ref,
              wt1_ref, b1_ref, g1_ref, be1_ref,
              ws1_ref, bs1_ref, gs1_ref, bes1_ref,
              wt2_ref, b2_ref, g2_ref, be2_ref,
              ws2_ref, bs2_ref, gs2_ref, bes2_ref,
              wt3_ref, b3_ref, g3_ref, be3_ref,
              ws3_ref, bs3_ref, gs3_ref, bes3_ref,
              out_ref, ha_ref, zs_ref):
    a = a_ref[...]
    cin1 = x_ref.shape[1]
    cmid = ha_ref.shape[1]

    def rd_x(lo, m):
        return (x_ref[pl.ds(lo, m)].astype(jnp.bfloat16)
                .reshape(m * cin1, L))

    def rd_h(lo, m):
        return ha_ref[pl.ds(lo, m)].reshape(m * cmid, L)

    _one_stage(G, B, cin1, a, rd_x, ha_ref,
               wt1_ref, b1_ref, g1_ref, be1_ref,
               ws1_ref, bs1_ref, gs1_ref, bes1_ref, zs_ref)
    _one_stage(G, B, cmid, a, rd_h, ha_ref,
               wt2_ref, b2_ref, g2_ref, be2_ref,
               ws2_ref, bs2_ref, gs2_ref, bes2_ref, zs_ref)
    _one_stage(G, B, cmid, a, rd_h, out_ref,
               wt3_ref, b3_ref, g3_ref, be3_ref,
               ws3_ref, bs3_ref, gs3_ref, bes3_ref, zs_ref)


def _fwd(x, a, params):
    G = x.shape[0]
    B = 4
    cout = 256
    return pl.pallas_call(
        functools.partial(_fwd_body, G, B),
        out_shape=jax.ShapeDtypeStruct((G, cout, L), jnp.float32),
        scratch_shapes=[pltpu.VMEM((G, cout, L), jnp.bfloat16),
                        pltpu.VMEM((B, cout, L), jnp.float32)],
    )(x, a, *params)


def _col(v):
    return v.reshape(-1, 1)


def kernel(x, edge_index, W1, b1, g1, be1, W1s, b1s, g1s, be1s,
           W2, b2, g2, be2, W2s, b2s, g2s, be2s,
           W3, b3, g3, be3, W3s, b3s, g3s, be3s):
    ei = edge_index.astype(jnp.int32)
    counts2, degp = _sc_counts(ei[0], ei[1])
    a = _prep(counts2.reshape(NC, L, L), degp.reshape(NC, L, 1))

    params = (W1.T, _col(b1), _col(g1), _col(be1),
              W1s.T, _col(b1s), _col(g1s), _col(be1s),
              W2.T, _col(b2), _col(g2), _col(be2),
              W2s.T, _col(b2s), _col(g2s), _col(be2s),
              W3.T, _col(b3), _col(g3), _col(be3),
              W3s.T, _col(b3s), _col(g3s), _col(be3s))
    return _fwd(x.reshape(16, x.shape[2], L), a, params)
